# baseline (speedup 1.0000x reference)
import functools

import numpy as np
import jax
import jax.numpy as jnp
from jax import lax
from jax.experimental import pallas as pl
from jax.experimental.pallas import tpu as pltpu
from jax.experimental.pallas import tpu_sc as plsc

DL = 10.0
SIGMA = 5.0
SIGMA_SQ_HALF = SIGMA ** 2 / 2.0
TWOPI = 2.0 * np.pi
TWOPI_SQ = (2.0 * np.pi) ** 2
K_SQ_MAX = (TWOPI / DL) ** 2
BOX_CONST = np.full((3,), 200.0, dtype=np.float32)

def _static_kpoints():
    nk = np.maximum((BOX_CONST / DL).astype(np.int32), 1)
    kx = np.arange(-int(nk[0]), int(nk[0]) + 1, dtype=np.int32)
    ky = np.arange(-int(nk[1]), int(nk[1]) + 1, dtype=np.int32)
    kz = np.arange(-int(nk[2]), int(nk[2]) + 1, dtype=np.int32)
    kxt = (kx.astype(np.float32) / BOX_CONST[0]) ** 2
    kyt = (ky.astype(np.float32) / BOX_CONST[1]) ** 2
    kzt = (kz.astype(np.float32) / BOX_CONST[2]) ** 2
    ksq = np.float32(TWOPI_SQ) * (
        kxt[:, None, None] + kyt[None, :, None] + kzt[None, None, :]
    )
    mask = (ksq <= np.float32(K_SQ_MAX)) & (ksq > 0)
    ix, iy, iz = np.nonzero(mask)
    return (
        kx[ix].astype(np.float32),
        ky[iy].astype(np.float32),
        kz[iz].astype(np.float32),
    )


_KXF, _KYF, _KZF = _static_kpoints()
N_SEL = _KXF.shape[0]

NUM_CORES = 2
NUM_SUBCORES = 16
LANES = 16
NW = NUM_CORES * NUM_SUBCORES

VECS_PER_W = -(-N_SEL // (NW * LANES))
CHUNK = VECS_PER_W * LANES
N_PAD = CHUNK * NW
TAIL = N_SEL - (NW - 1) * CHUNK
PACK = 3 * CHUNK


def _pad(a, fill):
    out = np.full((N_PAD,), fill, dtype=np.float32)
    out[:N_SEL] = a
    return out


_KX_PAD = _pad(_KXF, 1.0)
_KY_PAD = _pad(_KYF, 0.0)
_KZ_PAD = _pad(_KZF, 0.0)

_PACKED = np.empty((NW, PACK), dtype=np.float32)
_PACKED[:, :CHUNK] = _KX_PAD.reshape(NW, CHUNK)
_PACKED[:, CHUNK : 2 * CHUNK] = _KY_PAD.reshape(NW, CHUNK)
_PACKED[:, 2 * CHUNK :] = _KZ_PAD.reshape(NW, CHUNK)
_PACKED = _PACKED.reshape(-1)

def _ewald_body(pk_hbm, box_hbm, vx_hbm, vy_hbm, vz_hbm, fac_hbm,
                pk_v, box_v, kv_v, fac_v, sem):
    wid = lax.axis_index("s") * NUM_CORES + lax.axis_index("c")
    base = pl.multiple_of(wid * CHUNK, 8)

    in_cp = pltpu.make_async_copy(
        pk_hbm.at[pl.ds(pl.multiple_of(wid * PACK, 8), PACK)], pk_v, sem)
    box_cp = pltpu.make_async_copy(box_hbm, box_v.at[pl.ds(0, 3)], sem)
    in_cp.start()
    box_cp.start()
    box_cp.wait()
    in_cp.wait()

    barr = box_v[...]
    inv = [TWOPI / jnp.broadcast_to(barr[t], (LANES,)) for t in range(3)]

    is_tail = wid == NW - 1
    outs = (vx_hbm, vy_hbm, vz_hbm)

    def _out_copies(n):
        cps = [pltpu.make_async_copy(kv_v.at[pl.ds(c * CHUNK, n)],
                                     outs[c].at[pl.ds(base, n)], sem)
               for c in range(3)]
        cps.append(pltpu.make_async_copy(fac_v.at[pl.ds(0, n)],
                                         fac_hbm.at[pl.ds(base, n)], sem))
        return cps

    for c in range(3):
        def _stage(j, _, c=c):
            s = pl.ds(pl.multiple_of(c * CHUNK + j * LANES, LANES), LANES)
            kv_v[s] = pk_v[s] * inv[c]
            return _

        lax.fori_loop(0, VECS_PER_W, _stage, None, unroll=4)

        @pl.when(jnp.logical_not(is_tail))
        def _start_full(c=c):
            _out_copies(CHUNK)[c].start()

        @pl.when(is_tail)
        def _start_tail(c=c):
            _out_copies(TAIL)[c].start()

    def _factor(j, _):
        o = pl.multiple_of(j * LANES, LANES)
        vx = kv_v[pl.ds(o, LANES)]
        vy = kv_v[pl.ds(CHUNK + o, LANES)]
        vz = kv_v[pl.ds(2 * CHUNK + o, LANES)]
        ksq = vx * vx + vy * vy + vz * vz
        fac_v[pl.ds(o, LANES)] = (TWOPI * jnp.exp(-SIGMA_SQ_HALF * ksq)) / ksq
        return _

    lax.fori_loop(0, VECS_PER_W, _factor, None, unroll=4)

    @pl.when(jnp.logical_not(is_tail))
    def _finish_full():
        cps = _out_copies(CHUNK)
        cps[3].start()
        for cp in cps:
            cp.wait()

    @pl.when(is_tail)
    def _finish_tail():
        cps = _out_copies(TAIL)
        cps[3].start()
        for cp in cps:
            cp.wait()


@functools.cache
def _build_sc_call():
    return pl.kernel(
        _ewald_body,
        out_type=[
            jax.ShapeDtypeStruct((N_SEL,), jnp.float32),
            jax.ShapeDtypeStruct((N_SEL,), jnp.float32),
            jax.ShapeDtypeStruct((N_SEL,), jnp.float32),
            jax.ShapeDtypeStruct((N_SEL,), jnp.float32),
        ],
        mesh=plsc.VectorSubcoreMesh(
            core_axis_name="c", subcore_axis_name="s",
            num_cores=NUM_CORES, num_subcores=NUM_SUBCORES,
        ),
        scratch_types=[
            pltpu.VMEM((PACK,), jnp.float32),
            pltpu.VMEM((LANES,), jnp.float32),
            pltpu.VMEM((PACK,), jnp.float32),
            pltpu.VMEM((CHUNK,), jnp.float32),
            pltpu.SemaphoreType.DMA,
        ],
    )


def kernel(r_raw, box):
    del r_raw
    vx, vy, vz, factor = _build_sc_call()(
        jnp.asarray(_PACKED), box.astype(jnp.float32))
    return (jnp.stack((vx, vy, vz), axis=-1), factor)


# speedup vs baseline: 1.0417x; 1.0417x over previous
import functools

import numpy as np
import jax
import jax.numpy as jnp
from jax import lax
from jax.experimental import pallas as pl
from jax.experimental.pallas import tpu as pltpu
from jax.experimental.pallas import tpu_sc as plsc

DL = 10.0
SIGMA = 5.0
SIGMA_SQ_HALF = SIGMA ** 2 / 2.0
TWOPI = 2.0 * np.pi
TWOPI_SQ = (2.0 * np.pi) ** 2
K_SQ_MAX = (TWOPI / DL) ** 2
BOX_CONST = np.full((3,), 200.0, dtype=np.float32)

def _static_kpoints():
    nk = np.maximum((BOX_CONST / DL).astype(np.int32), 1)
    kx = np.arange(-int(nk[0]), int(nk[0]) + 1, dtype=np.int32)
    ky = np.arange(-int(nk[1]), int(nk[1]) + 1, dtype=np.int32)
    kz = np.arange(-int(nk[2]), int(nk[2]) + 1, dtype=np.int32)
    kxt = (kx.astype(np.float32) / BOX_CONST[0]) ** 2
    kyt = (ky.astype(np.float32) / BOX_CONST[1]) ** 2
    kzt = (kz.astype(np.float32) / BOX_CONST[2]) ** 2
    ksq = np.float32(TWOPI_SQ) * (
        kxt[:, None, None] + kyt[None, :, None] + kzt[None, None, :]
    )
    mask = (ksq <= np.float32(K_SQ_MAX)) & (ksq > 0)
    ix, iy, iz = np.nonzero(mask)
    return (
        kx[ix].astype(np.float32),
        ky[iy].astype(np.float32),
        kz[iz].astype(np.float32),
    )


_KXF, _KYF, _KZF = _static_kpoints()
N_SEL = _KXF.shape[0]

NUM_CORES = 2
NUM_SUBCORES = 16
LANES = 16
NW = NUM_CORES * NUM_SUBCORES

VECS_PER_W = -(-N_SEL // (NW * LANES))
CHUNK = VECS_PER_W * LANES
N_PAD = CHUNK * NW
TAIL = N_SEL - (NW - 1) * CHUNK
PACK = 3 * CHUNK


def _pad(a, fill):
    out = np.full((N_PAD,), fill, dtype=np.float32)
    out[:N_SEL] = a
    return out


_KX_PAD = _pad(_KXF, 1.0)
_KY_PAD = _pad(_KYF, 0.0)
_KZ_PAD = _pad(_KZF, 0.0)

_PACKED = np.empty((NW, PACK), dtype=np.float32)
_PACKED[:, :CHUNK] = _KX_PAD.reshape(NW, CHUNK)
_PACKED[:, CHUNK : 2 * CHUNK] = _KY_PAD.reshape(NW, CHUNK)
_PACKED[:, 2 * CHUNK :] = _KZ_PAD.reshape(NW, CHUNK)
_PACKED = _PACKED.reshape(-1)

def _ewald_body(pk_hbm, box_hbm, vx_hbm, vy_hbm, vz_hbm, fac_hbm,
                pk_v, box_v, kv_v, fac_v, sem):
    wid = lax.axis_index("s") * NUM_CORES + lax.axis_index("c")
    base = pl.multiple_of(wid * CHUNK, 8)

    in_cp = pltpu.make_async_copy(
        pk_hbm.at[pl.ds(pl.multiple_of(wid * PACK, 8), PACK)], pk_v, sem)
    box_cp = pltpu.make_async_copy(box_hbm, box_v.at[pl.ds(0, 3)], sem)
    in_cp.start()
    box_cp.start()
    box_cp.wait()
    in_cp.wait()

    barr = box_v[...]
    inv = [TWOPI / jnp.broadcast_to(barr[t], (LANES,)) for t in range(3)]

    is_tail = wid == NW - 1
    outs = (vx_hbm, vy_hbm, vz_hbm)

    def _out_copies(n):
        cps = [pltpu.make_async_copy(kv_v.at[pl.ds(c * CHUNK, n)],
                                     outs[c].at[pl.ds(base, n)], sem)
               for c in range(3)]
        cps.append(pltpu.make_async_copy(fac_v.at[pl.ds(0, n)],
                                         fac_hbm.at[pl.ds(base, n)], sem))
        return cps

    for c in range(3):
        for j in range(VECS_PER_W):
            s = pl.ds(c * CHUNK + j * LANES, LANES)
            kv_v[s] = pk_v[s] * inv[c]

        @pl.when(jnp.logical_not(is_tail))
        def _start_full(c=c):
            _out_copies(CHUNK)[c].start()

        @pl.when(is_tail)
        def _start_tail(c=c):
            _out_copies(TAIL)[c].start()

    for j in range(VECS_PER_W):
        s = pl.ds(j * LANES, LANES)
        vx = kv_v[pl.ds(j * LANES, LANES)]
        vy = kv_v[pl.ds(CHUNK + j * LANES, LANES)]
        vz = kv_v[pl.ds(2 * CHUNK + j * LANES, LANES)]
        ksq = vx * vx + vy * vy + vz * vz
        fac_v[s] = (TWOPI * jnp.exp(-SIGMA_SQ_HALF * ksq)) / ksq

    @pl.when(jnp.logical_not(is_tail))
    def _finish_full():
        cps = _out_copies(CHUNK)
        cps[3].start()
        for cp in cps:
            cp.wait()

    @pl.when(is_tail)
    def _finish_tail():
        cps = _out_copies(TAIL)
        cps[3].start()
        for cp in cps:
            cp.wait()


@functools.cache
def _build_sc_call():
    return pl.kernel(
        _ewald_body,
        out_type=[
            jax.ShapeDtypeStruct((N_SEL,), jnp.float32),
            jax.ShapeDtypeStruct((N_SEL,), jnp.float32),
            jax.ShapeDtypeStruct((N_SEL,), jnp.float32),
            jax.ShapeDtypeStruct((N_SEL,), jnp.float32),
        ],
        mesh=plsc.VectorSubcoreMesh(
            core_axis_name="c", subcore_axis_name="s",
            num_cores=NUM_CORES, num_subcores=NUM_SUBCORES,
        ),
        scratch_types=[
            pltpu.VMEM((PACK,), jnp.float32),
            pltpu.VMEM((LANES,), jnp.float32),
            pltpu.VMEM((PACK,), jnp.float32),
            pltpu.VMEM((CHUNK,), jnp.float32),
            pltpu.SemaphoreType.DMA,
        ],
    )


def kernel(r_raw, box):
    del r_raw
    vx, vy, vz, factor = _build_sc_call()(
        jnp.asarray(_PACKED), box.astype(jnp.float32))
    return (jnp.stack((vx, vy, vz), axis=-1), factor)
